# baseline (speedup 1.0000x reference)
import jax
import jax.numpy as jnp
from jax.experimental import pallas as pl
from jax.experimental.pallas import tpu as pltpu

_NCHUNK = 9


def _body(x_hbm, p_hbm, out_hbm, xbuf, pbuf, segbuf, ostage, xsem, psem, osem):
    b = pl.program_id(0)
    C, N = xbuf.shape
    K = pbuf.shape[0]
    nb = N // _NCHUNK

    pltpu.make_async_copy(p_hbm.at[b], pbuf, psem).start()
    for j in range(_NCHUNK):
        sl = pl.ds(j * nb, nb)
        pltpu.make_async_copy(x_hbm.at[b, :, sl], xbuf.at[:, sl],
                              xsem.at[j]).start()

    pltpu.make_async_copy(p_hbm.at[b], pbuf, psem).wait()

    num = jnp.zeros((K, C), jnp.float32)
    den = jnp.zeros((K, 1), jnp.float32)
    cnt = jnp.zeros((K, 1), jnp.float32)
    for j in range(_NCHUNK):
        sl = pl.ds(j * nb, nb)
        p = pbuf[:, sl]
        s = jnp.max(p, axis=0, keepdims=True)
        iota_k = jax.lax.broadcasted_iota(jnp.int32, (K, nb), 0)
        seg = jnp.min(jnp.where(p == s, iota_k, K), axis=0, keepdims=True)
        segbuf[:, sl] = seg
        e = jnp.exp(s)
        onehot = iota_k == seg
        onehot_e = jnp.where(onehot, jnp.broadcast_to(e, (K, nb)), 0.0)
        pltpu.make_async_copy(x_hbm.at[b, :, sl], xbuf.at[:, sl],
                              xsem.at[j]).wait()
        feats = xbuf[:, sl]
        nt = (((1,), (1,)), ((), ()))
        num = num + jax.lax.dot_general(onehot_e, feats, nt,
                                        preferred_element_type=jnp.float32)
        den = den + jnp.sum(onehot_e, axis=1, keepdims=True)
        cnt = cnt + jnp.sum(jnp.where(onehot, 1.0, 0.0), axis=1, keepdims=True)

    den_safe = jnp.where(den > 0, den, 1.0)
    scale = jnp.where(cnt > 0, 1.0 / (den_safe * jnp.maximum(cnt, 1.0)), 0.0)
    resultT = num * scale

    for j in range(_NCHUNK):
        sl = pl.ds(j * nb, nb)
        slot = j % 2
        seg = segbuf[:, sl]
        iota_k = jax.lax.broadcasted_iota(jnp.int32, (K, nb), 0)
        onehot_f = jnp.where(iota_k == seg, 1.0, 0.0)
        tn = (((0,), (0,)), ((), ()))
        feats_sl = jax.lax.dot_general(resultT, onehot_f, tn,
                                       preferred_element_type=jnp.float32)
        if j >= 2:
            pltpu.make_async_copy(
                ostage.at[slot], out_hbm.at[b, :, pl.ds((j - 2) * nb, nb)],
                osem.at[slot]).wait()
        ostage[slot] = xbuf[:, sl] * feats_sl
        pltpu.make_async_copy(ostage.at[slot], out_hbm.at[b, :, sl],
                              osem.at[slot]).start()
    for j in range(_NCHUNK - 2, _NCHUNK):
        slot = j % 2
        pltpu.make_async_copy(ostage.at[slot], out_hbm.at[b, :, pl.ds(j * nb, nb)],
                              osem.at[slot]).wait()


@jax.jit
def kernel(x, preds):
    B, C, h, w, d = x.shape
    K = preds.shape[1]
    N = h * w * d
    nb = N // _NCHUNK
    assert N % _NCHUNK == 0
    xr = x.reshape(B, C, N)
    pr = preds.reshape(B, K, N)

    out = pl.pallas_call(
        _body,
        grid=(B,),
        in_specs=[
            pl.BlockSpec(memory_space=pl.ANY),
            pl.BlockSpec(memory_space=pl.ANY),
        ],
        out_specs=pl.BlockSpec(memory_space=pl.ANY),
        out_shape=jax.ShapeDtypeStruct((B, C, N), jnp.float32),
        scratch_shapes=[
            pltpu.VMEM((C, N), jnp.float32),
            pltpu.VMEM((K, N), jnp.float32),
            pltpu.VMEM((1, N), jnp.int32),
            pltpu.VMEM((2, C, nb), jnp.float32),
            pltpu.SemaphoreType.DMA((_NCHUNK,)),
            pltpu.SemaphoreType.DMA,
            pltpu.SemaphoreType.DMA((2,)),
        ],
        compiler_params=pltpu.CompilerParams(
            dimension_semantics=("arbitrary",)),
    )(xr, pr)

    return out.reshape(B, C, h, w, d)


# speedup vs baseline: 1.0218x; 1.0218x over previous
import jax
import jax.numpy as jnp
from jax.experimental import pallas as pl
from jax.experimental.pallas import tpu as pltpu

_NCHUNK = 9


def _body(x_hbm, p_hbm, out_hbm, xbuf, pbuf, segbuf, ostage, xsem, psem, osem):
    b = pl.program_id(0)
    B = pl.num_programs(0)
    C, N = xbuf.shape
    K = pbuf.shape[0]
    nb = N // _NCHUNK

    @pl.when(b == 0)
    def _():
        pltpu.make_async_copy(p_hbm.at[0], pbuf, psem).start()
        for j in range(_NCHUNK):
            sl = pl.ds(j * nb, nb)
            pltpu.make_async_copy(x_hbm.at[0, :, sl], xbuf.at[:, sl],
                                  xsem.at[j]).start()

    pltpu.make_async_copy(p_hbm.at[b], pbuf, psem).wait()

    num = jnp.zeros((K, C), jnp.float32)
    den = jnp.zeros((K, 1), jnp.float32)
    cnt = jnp.zeros((K, 1), jnp.float32)
    for j in range(_NCHUNK):
        sl = pl.ds(j * nb, nb)
        p = pbuf[:, sl]
        s = jnp.max(p, axis=0, keepdims=True)
        iota_k = jax.lax.broadcasted_iota(jnp.int32, (K, nb), 0)
        seg = jnp.min(jnp.where(p == s, iota_k, K), axis=0, keepdims=True)
        segbuf[:, sl] = seg
        e = jnp.exp(s)
        onehot = iota_k == seg
        onehot_e = jnp.where(onehot, jnp.broadcast_to(e, (K, nb)), 0.0)
        pltpu.make_async_copy(x_hbm.at[b, :, sl], xbuf.at[:, sl],
                              xsem.at[j]).wait()
        feats = xbuf[:, sl]
        nt = (((1,), (1,)), ((), ()))
        num = num + jax.lax.dot_general(onehot_e, feats, nt,
                                        preferred_element_type=jnp.float32)
        den = den + jnp.sum(onehot_e, axis=1, keepdims=True)
        cnt = cnt + jnp.sum(jnp.where(onehot, 1.0, 0.0), axis=1, keepdims=True)

    den_safe = jnp.where(den > 0, den, 1.0)
    scale = jnp.where(cnt > 0, 1.0 / (den_safe * jnp.maximum(cnt, 1.0)), 0.0)
    resultT = num * scale

    @pl.when(b + 1 < B)
    def _():
        pltpu.make_async_copy(p_hbm.at[b + 1], pbuf, psem).start()

    for j in range(_NCHUNK):
        sl = pl.ds(j * nb, nb)
        slot = j % 2
        seg = segbuf[:, sl]
        iota_k = jax.lax.broadcasted_iota(jnp.int32, (K, nb), 0)
        onehot_f = jnp.where(iota_k == seg, 1.0, 0.0)
        tn = (((0,), (0,)), ((), ()))
        feats_sl = jax.lax.dot_general(resultT, onehot_f, tn,
                                       preferred_element_type=jnp.float32)
        if j >= 2:
            pltpu.make_async_copy(
                ostage.at[slot], out_hbm.at[b, :, pl.ds((j - 2) * nb, nb)],
                osem.at[slot]).wait()
        ostage[slot] = xbuf[:, sl] * feats_sl
        pltpu.make_async_copy(ostage.at[slot], out_hbm.at[b, :, sl],
                              osem.at[slot]).start()

        @pl.when(b + 1 < B)
        def _():
            pltpu.make_async_copy(x_hbm.at[b + 1, :, sl], xbuf.at[:, sl],
                                  xsem.at[j]).start()

    for j in range(_NCHUNK - 2, _NCHUNK):
        slot = j % 2
        pltpu.make_async_copy(ostage.at[slot], out_hbm.at[b, :, pl.ds(j * nb, nb)],
                              osem.at[slot]).wait()


@jax.jit
def kernel(x, preds):
    B, C, h, w, d = x.shape
    K = preds.shape[1]
    N = h * w * d
    nb = N // _NCHUNK
    assert N % _NCHUNK == 0
    xr = x.reshape(B, C, N)
    pr = preds.reshape(B, K, N)

    out = pl.pallas_call(
        _body,
        grid=(B,),
        in_specs=[
            pl.BlockSpec(memory_space=pl.ANY),
            pl.BlockSpec(memory_space=pl.ANY),
        ],
        out_specs=pl.BlockSpec(memory_space=pl.ANY),
        out_shape=jax.ShapeDtypeStruct((B, C, N), jnp.float32),
        scratch_shapes=[
            pltpu.VMEM((C, N), jnp.float32),
            pltpu.VMEM((K, N), jnp.float32),
            pltpu.VMEM((1, N), jnp.int32),
            pltpu.VMEM((2, C, nb), jnp.float32),
            pltpu.SemaphoreType.DMA((_NCHUNK,)),
            pltpu.SemaphoreType.DMA,
            pltpu.SemaphoreType.DMA((2,)),
        ],
        compiler_params=pltpu.CompilerParams(
            dimension_semantics=("arbitrary",)),
    )(xr, pr)

    return out.reshape(B, C, h, w, d)
